# Initial kernel scaffold; baseline (speedup 1.0000x reference)
import functools

import jax
import jax.numpy as jnp
from jax import lax
from jax.experimental import pallas as pl
from jax.experimental.pallas import tpu as pltpu
from jax.experimental.pallas import tpu_sc as plsc

_N = 10000
_E = 320000
_D = 128
_DO = 64

_NC = 2
_NS = 16
_NW = _NC * _NS
_CHUNK = 128
_NCHUNK = -(-_E // (_NW * _CHUNK))
_EPW = _NCHUNK * _CHUNK
_EPAD = _EPW * _NW
_RPT = _N // _NS

_BM = 1000


def _build_spmm():
    mesh = plsc.VectorSubcoreMesh(core_axis_name="c", subcore_axis_name="s")

    @functools.partial(
        pl.kernel,
        out_type=jax.ShapeDtypeStruct((_NC, _N, _D), jnp.float32),
        mesh=mesh,
        scratch_types=[
            pltpu.VMEM((_NCHUNK, _CHUNK), jnp.int32),
            pltpu.VMEM((_NCHUNK, _CHUNK), jnp.int32),
            pltpu.VMEM((_NCHUNK, _CHUNK), jnp.float32),
            pltpu.VMEM((_CHUNK, _D), jnp.float32),
            pltpu.VMEM_SHARED((_N, _D), jnp.float32),
            pltpu.SemaphoreType.DMA,
        ],
    )
    def spmm(h_hbm, src_hbm, dst_hbm, w_hbm, z_hbm, out_hbm,
             src_v, dst_v, w_v, rows_v, acc_sh, sem):
        cid = lax.axis_index("c")
        sid = lax.axis_index("s")
        wid = cid * _NS + sid
        r0 = sid * _RPT

        pltpu.sync_copy(z_hbm.at[pl.ds(r0, _RPT)], acc_sh.at[pl.ds(r0, _RPT)])
        pltpu.sync_copy(src_hbm.at[wid], src_v)
        pltpu.sync_copy(dst_hbm.at[wid], dst_v)
        pltpu.sync_copy(w_hbm.at[wid], w_v)
        plsc.subcore_barrier()

        def chunk_body(c, carry):
            pltpu.async_copy(h_hbm.at[src_v.at[c]], rows_v, sem).wait()

            def edge_body(e, carry2):
                w = w_v[c, e]
                for j in range(_D // 16):
                    sl = pl.ds(j * 16, 16)
                    rows_v[e, sl] = rows_v[e, sl] * w
                return carry2

            lax.fori_loop(0, _CHUNK, edge_body, 0, unroll=2)
            pltpu.sync_copy(rows_v, acc_sh.at[dst_v.at[c]], add=True)
            return carry

        lax.fori_loop(0, _NCHUNK, chunk_body, 0)
        plsc.subcore_barrier()
        pltpu.sync_copy(acc_sh.at[pl.ds(r0, _RPT)],
                        out_hbm.at[cid, pl.ds(r0, _RPT)])

    return spmm


_spmm = _build_spmm()


def _mm_body(x_ref, w_ref, o_ref):
    o_ref[...] = jnp.dot(x_ref[...], w_ref[...],
                         preferred_element_type=jnp.float32)


def _mm(x, w):
    n, k = x.shape
    m = w.shape[1]
    return pl.pallas_call(
        _mm_body,
        grid=(n // _BM,),
        in_specs=[
            pl.BlockSpec((_BM, k), lambda i: (i, 0)),
            pl.BlockSpec((k, m), lambda i: (0, 0)),
        ],
        out_specs=pl.BlockSpec((_BM, m), lambda i: (i, 0)),
        out_shape=jax.ShapeDtypeStruct((n, m), jnp.float32),
    )(x, w)


def _fuse_body(p_ref, w_ref, o_ref):
    h = jnp.maximum(p_ref[0] + p_ref[1], 0.0)
    o_ref[...] = jnp.dot(h, w_ref[...], preferred_element_type=jnp.float32)


def _fuse_relu_mm(p, w):
    k, m = w.shape
    return pl.pallas_call(
        _fuse_body,
        grid=(_N // _BM,),
        in_specs=[
            pl.BlockSpec((_NC, _BM, k), lambda i: (0, i, 0)),
            pl.BlockSpec((k, m), lambda i: (0, 0)),
        ],
        out_specs=pl.BlockSpec((_BM, m), lambda i: (i, 0)),
        out_shape=jax.ShapeDtypeStruct((_N, m), jnp.float32),
    )(p, w)


def _fin_body(q_ref, mu_ref, lv_ref):
    s = q_ref[0] + q_ref[1]
    m = s[:, :_DO]
    norm = jnp.sqrt(jnp.sum(m * m, axis=1, keepdims=True))
    mu_ref[...] = m / jnp.maximum(norm, 1e-12)
    lv_ref[...] = s[:, _DO:]


def _finalize(q):
    return pl.pallas_call(
        _fin_body,
        grid=(_N // _BM,),
        in_specs=[pl.BlockSpec((_NC, _BM, _D), lambda i: (0, i, 0))],
        out_specs=[
            pl.BlockSpec((_BM, _DO), lambda i: (i, 0)),
            pl.BlockSpec((_BM, _DO), lambda i: (i, 0)),
        ],
        out_shape=[
            jax.ShapeDtypeStruct((_N, _DO), jnp.float32),
            jax.ShapeDtypeStruct((_N, _DO), jnp.float32),
        ],
    )(q)


def kernel(x, adj, edge_weight, W1, W2, W3):
    pad = _EPAD - _E
    src = jnp.concatenate([adj[0], jnp.zeros((pad,), jnp.int32)])
    dst = jnp.concatenate([adj[1], jnp.zeros((pad,), jnp.int32)])
    ew = jnp.concatenate([edge_weight, jnp.zeros((pad,), jnp.float32)])
    src = src.reshape(_NW, _NCHUNK, _CHUNK)
    dst = dst.reshape(_NW, _NCHUNK, _CHUNK)
    ew = ew.reshape(_NW, _NCHUNK, _CHUNK)
    zeros = jnp.zeros((_N, _D), jnp.float32)
    wcat = jnp.concatenate([W2, W3], axis=1)

    xw = _mm(x, W1)
    p = _spmm(xw, src, dst, ew, zeros)
    hw = _fuse_relu_mm(p, wcat)
    q = _spmm(hw, src, dst, ew, zeros)
    mu, logvar = _finalize(q)
    return (mu, mu, logvar)


# speedup vs baseline: 6.0874x; 6.0874x over previous
import functools

import jax
import jax.numpy as jnp
from jax import lax
from jax.experimental import pallas as pl
from jax.experimental.pallas import tpu as pltpu
from jax.experimental.pallas import tpu_sc as plsc

_N = 10000
_E = 320000
_D = 128
_DO = 64

_NC = 2
_NS = 16
_NW = _NC * _NS
_CHUNK = 128
_NCHUNK = -(-_E // (_NW * _CHUNK))
_EPW = _NCHUNK * _CHUNK
_EPAD = _EPW * _NW
_NPAD = 10240
_RPT = _NPAD // _NS

_BM = 1000


def _build_spmm():
    mesh = plsc.VectorSubcoreMesh(core_axis_name="c", subcore_axis_name="s")

    @functools.partial(
        pl.kernel,
        out_type=jax.ShapeDtypeStruct((_NC, _NPAD, _D), jnp.float32),
        mesh=mesh,
        scratch_types=[
            pltpu.VMEM((_NCHUNK, _CHUNK), jnp.int32),
            pltpu.VMEM((_NCHUNK, _CHUNK), jnp.int32),
            pltpu.VMEM((_NCHUNK, _CHUNK), jnp.float32),
            pltpu.VMEM((_CHUNK, _D), jnp.float32),
            pltpu.VMEM_SHARED((_NPAD, _D), jnp.float32),
            pltpu.SemaphoreType.DMA,
        ],
    )
    def spmm(h_hbm, src_hbm, dst_hbm, w_hbm, z_hbm, out_hbm,
             src_v, dst_v, w_v, rows_v, acc_sh, sem):
        cid = lax.axis_index("c")
        sid = lax.axis_index("s")
        wid = cid * _NS + sid
        r0 = sid * _RPT

        pltpu.sync_copy(z_hbm.at[pl.ds(r0, _RPT)], acc_sh.at[pl.ds(r0, _RPT)])
        pltpu.sync_copy(src_hbm.at[wid], src_v)
        pltpu.sync_copy(dst_hbm.at[wid], dst_v)
        pltpu.sync_copy(w_hbm.at[wid], w_v)
        plsc.subcore_barrier()

        def chunk_body(c, carry):
            pltpu.async_copy(h_hbm.at[src_v.at[c]], rows_v, sem).wait()

            def group_body(g, carry2):
                wv = w_v[c, pl.ds(g * 16, 16)]
                base = g * 16
                for t in range(16):
                    w = wv[t]
                    for j in range(_D // 16):
                        sl = pl.ds(j * 16, 16)
                        rows_v[base + t, sl] = rows_v[base + t, sl] * w
                return carry2

            lax.fori_loop(0, _CHUNK // 16, group_body, 0)
            pltpu.sync_copy(rows_v, acc_sh.at[dst_v.at[c]], add=True)
            return carry

        lax.fori_loop(0, _NCHUNK, chunk_body, 0)
        plsc.subcore_barrier()
        pltpu.sync_copy(acc_sh.at[pl.ds(r0, _RPT)],
                        out_hbm.at[cid, pl.ds(r0, _RPT)])

    return spmm


_spmm = _build_spmm()


def _mm_body(x_ref, w_ref, o_ref):
    o_ref[...] = jnp.dot(x_ref[...], w_ref[...],
                         preferred_element_type=jnp.float32)


def _mm(x, w):
    n, k = x.shape
    m = w.shape[1]
    return pl.pallas_call(
        _mm_body,
        grid=(n // _BM,),
        in_specs=[
            pl.BlockSpec((_BM, k), lambda i: (i, 0)),
            pl.BlockSpec((k, m), lambda i: (0, 0)),
        ],
        out_specs=pl.BlockSpec((_BM, m), lambda i: (i, 0)),
        out_shape=jax.ShapeDtypeStruct((n, m), jnp.float32),
    )(x, w)


def _fuse_body(p_ref, w_ref, o_ref):
    h = jnp.maximum(p_ref[0] + p_ref[1], 0.0)
    o_ref[...] = jnp.dot(h, w_ref[...], preferred_element_type=jnp.float32)


def _fuse_relu_mm(p, w):
    k, m = w.shape
    return pl.pallas_call(
        _fuse_body,
        grid=(_N // _BM,),
        in_specs=[
            pl.BlockSpec((_NC, _BM, k), lambda i: (0, i, 0)),
            pl.BlockSpec((k, m), lambda i: (0, 0)),
        ],
        out_specs=pl.BlockSpec((_BM, m), lambda i: (i, 0)),
        out_shape=jax.ShapeDtypeStruct((_N, m), jnp.float32),
    )(p, w)


def _fin_body(q_ref, mu_ref, lv_ref):
    s = q_ref[0] + q_ref[1]
    m = s[:, :_DO]
    norm = jnp.sqrt(jnp.sum(m * m, axis=1, keepdims=True))
    mu_ref[...] = m / jnp.maximum(norm, 1e-12)
    lv_ref[...] = s[:, _DO:]


def _finalize(q):
    return pl.pallas_call(
        _fin_body,
        grid=(_N // _BM,),
        in_specs=[pl.BlockSpec((_NC, _BM, _D), lambda i: (0, i, 0))],
        out_specs=[
            pl.BlockSpec((_BM, _DO), lambda i: (i, 0)),
            pl.BlockSpec((_BM, _DO), lambda i: (i, 0)),
        ],
        out_shape=[
            jax.ShapeDtypeStruct((_N, _DO), jnp.float32),
            jax.ShapeDtypeStruct((_N, _DO), jnp.float32),
        ],
    )(q)


def kernel(x, adj, edge_weight, W1, W2, W3):
    pad = _EPAD - _E
    src = jnp.concatenate([adj[0], jnp.zeros((pad,), jnp.int32)])
    dst = jnp.concatenate([adj[1], jnp.zeros((pad,), jnp.int32)])
    ew = jnp.concatenate([edge_weight, jnp.zeros((pad,), jnp.float32)])
    src = src.reshape(_NW, _NCHUNK, _CHUNK)
    dst = dst.reshape(_NW, _NCHUNK, _CHUNK)
    ew = ew.reshape(_NW, _NCHUNK, _CHUNK)
    zeros = jnp.zeros((_NPAD, _D), jnp.float32)
    wcat = jnp.concatenate([W2, W3], axis=1)

    xw = _mm(x, W1)
    p = _spmm(xw, src, dst, ew, zeros)
    hw = _fuse_relu_mm(p, wcat)
    q = _spmm(hw, src, dst, ew, zeros)
    mu, logvar = _finalize(q)
    return (mu, mu, logvar)
